# Initial kernel scaffold; baseline (speedup 1.0000x reference)
import jax
import jax.numpy as jnp
from jax.experimental import pallas as pl


def kernel(x, edge_index, edge_type, weight, w_comp, self_loop_weight):
    raise NotImplementedError("write your pallas kernel here")


# speedup vs baseline: 22.5006x; 22.5006x over previous
import functools

import jax
import jax.numpy as jnp
from jax import lax
from jax.experimental import pallas as pl
from jax.experimental.pallas import tpu as pltpu
from jax.experimental.pallas import tpu_sc as plsc

N = 10000
E = 160000
IN_DIM = 256
OUT_DIM = 256
NUM_RELS = 16
NUM_BASES = 4

NC = 2
NS = 16
HALF = OUT_DIM // 2

BN = 400
NB = N // BN

E_PER = E // NS
CH = 80
NCHUNK = E_PER // CH

WB_TILES = 10
WB_ROWS = N // WB_TILES


def _table_kernel(wc_ref, x_ref, w_ref, out_ref):
    r = pl.program_id(1)
    w = (wc_ref[r, 0] * w_ref[0] + wc_ref[r, 1] * w_ref[1]
         + wc_ref[r, 2] * w_ref[2] + wc_ref[r, 3] * w_ref[3])
    zz = jnp.dot(x_ref[...], w, preferred_element_type=jnp.float32)
    out_ref[0] = zz[:, :HALF]
    out_ref[1] = zz[:, HALF:]


def _build_table(x, weight, w_comp):
    return pl.pallas_call(
        _table_kernel,
        grid=(NB, NUM_RELS),
        in_specs=[
            pl.BlockSpec(memory_space=pltpu.SMEM),
            pl.BlockSpec((BN, IN_DIM), lambda nb, r: (nb, 0)),
            pl.BlockSpec((NUM_BASES, IN_DIM, OUT_DIM), lambda nb, r: (0, 0, 0)),
        ],
        out_specs=pl.BlockSpec((NC, BN, HALF), lambda nb, r: (r, nb, 0)),
        out_shape=jax.ShapeDtypeStruct((NUM_RELS * NC, N, HALF), jnp.float32),
    )(w_comp, x, weight)


def _sc_body(table_hbm, src_hbm, etype_hbm, dst_hbm, zeros_hbm, out_hbm,
             src_buf, etype_buf, dst_buf, idx_buf, rows, agg, sem):
    c = lax.axis_index("c")
    s = lax.axis_index("s")

    @pl.when(s < WB_TILES)
    def _():
        pltpu.sync_copy(zeros_hbm, agg.at[pl.ds(s * WB_ROWS, WB_ROWS)])
    plsc.subcore_barrier()

    ebase = s * E_PER

    def chunk(i, _):
        eoff = pl.multiple_of(ebase + i * CH, 8)
        pltpu.sync_copy(src_hbm.at[pl.ds(eoff, CH)], src_buf)
        pltpu.sync_copy(etype_hbm.at[pl.ds(eoff, CH)], etype_buf)
        pltpu.sync_copy(dst_hbm.at[pl.ds(eoff, CH)], dst_buf)
        for k in range(CH // 16):
            sl = pl.ds(k * 16, 16)
            idx_buf[sl] = (etype_buf[sl] * NC + c) * N + src_buf[sl]
        pltpu.async_copy(table_hbm.at[idx_buf], rows, sem).wait()
        pltpu.sync_copy(rows, agg.at[dst_buf], add=True)
        return ()

    lax.fori_loop(0, NCHUNK, chunk, (), unroll=False)
    plsc.subcore_barrier()

    @pl.when(s < WB_TILES)
    def _():
        sl_out = pl.ds(s * WB_ROWS, WB_ROWS)
        pltpu.sync_copy(agg.at[sl_out], out_hbm.at[c, sl_out])


def _sc_aggregate(table, src, etype, dst, zeros):
    mesh = plsc.VectorSubcoreMesh(core_axis_name="c", subcore_axis_name="s")
    kern = pl.kernel(
        _sc_body,
        out_type=jax.ShapeDtypeStruct((NC, N, HALF), jnp.float32),
        mesh=mesh,
        scratch_types=[
            pltpu.VMEM((CH,), jnp.int32),
            pltpu.VMEM((CH,), jnp.int32),
            pltpu.VMEM((CH,), jnp.int32),
            pltpu.VMEM((CH,), jnp.int32),
            pltpu.VMEM((CH, HALF), jnp.float32),
            pltpu.VMEM_SHARED((N, HALF), jnp.float32),
            pltpu.SemaphoreType.DMA,
        ],
    )
    return kern(table, src, etype, dst, zeros)


def _final_kernel(x_ref, slw_ref, agg_ref, out_ref):
    h = jnp.dot(x_ref[...], slw_ref[...], preferred_element_type=jnp.float32)
    out_ref[...] = h + jnp.concatenate([agg_ref[0], agg_ref[1]], axis=1)


def _finalize(x, self_loop_weight, agg):
    bn = 1000
    return pl.pallas_call(
        _final_kernel,
        grid=(N // bn,),
        in_specs=[
            pl.BlockSpec((bn, IN_DIM), lambda i: (i, 0)),
            pl.BlockSpec((IN_DIM, OUT_DIM), lambda i: (0, 0)),
            pl.BlockSpec((NC, bn, HALF), lambda i: (0, i, 0)),
        ],
        out_specs=pl.BlockSpec((bn, OUT_DIM), lambda i: (i, 0)),
        out_shape=jax.ShapeDtypeStruct((N, OUT_DIM), jnp.float32),
    )(x, self_loop_weight, agg)


def kernel(x, edge_index, edge_type, weight, w_comp, self_loop_weight):
    src = edge_index[0]
    dst = edge_index[1]
    zeros = jnp.zeros((WB_ROWS, HALF), jnp.float32)

    table = _build_table(x, weight, w_comp)
    agg = _sc_aggregate(table.reshape(NUM_RELS * NC * N, HALF),
                        src, edge_type, dst, zeros)
    return _finalize(x, self_loop_weight, agg)


# speedup vs baseline: 58.8337x; 2.6148x over previous
import jax
import jax.numpy as jnp
from jax import lax
from jax.experimental import pallas as pl
from jax.experimental.pallas import tpu as pltpu
from jax.experimental.pallas import tpu_sc as plsc

N = 10000
E = 160000
IN_DIM = 256
OUT_DIM = 256
NUM_RELS = 16
NUM_BASES = 4

NC = 2
NS = 16
HALF = OUT_DIM // 2

BN = 2000
NB = N // BN

E_PER = E // NS
CH = 80
NCHUNK = E_PER // CH

WB_TILES = 10
WB_ROWS = N // WB_TILES


def _table_kernel(wc_ref, x_ref, w_ref, slw_ref, out_ref, curr_ref):
    r = pl.program_id(1)
    w = (wc_ref[r, 0] * w_ref[0] + wc_ref[r, 1] * w_ref[1]
         + wc_ref[r, 2] * w_ref[2] + wc_ref[r, 3] * w_ref[3])
    zz = jnp.dot(x_ref[...], w, preferred_element_type=jnp.float32)
    out_ref[0] = zz[:, :HALF]
    out_ref[1] = zz[:, HALF:]

    @pl.when(r == 0)
    def _():
        curr_ref[...] = jnp.dot(x_ref[...], slw_ref[...],
                                preferred_element_type=jnp.float32)


def _build_table(x, weight, w_comp, self_loop_weight):
    return pl.pallas_call(
        _table_kernel,
        grid=(NB, NUM_RELS),
        in_specs=[
            pl.BlockSpec(memory_space=pltpu.SMEM),
            pl.BlockSpec((BN, IN_DIM), lambda nb, r: (nb, 0)),
            pl.BlockSpec((NUM_BASES, IN_DIM, OUT_DIM), lambda nb, r: (0, 0, 0)),
            pl.BlockSpec((IN_DIM, OUT_DIM), lambda nb, r: (0, 0)),
        ],
        out_specs=[
            pl.BlockSpec((NC, BN, HALF), lambda nb, r: (r, nb, 0)),
            pl.BlockSpec((BN, OUT_DIM), lambda nb, r: (nb, 0)),
        ],
        out_shape=[
            jax.ShapeDtypeStruct((NUM_RELS * NC, N, HALF), jnp.float32),
            jax.ShapeDtypeStruct((N, OUT_DIM), jnp.float32),
        ],
    )(w_comp, x, weight, self_loop_weight)


def _sc_body(table_hbm, packed_hbm, zeros_hbm, out_hbm,
             ebufs, idxbufs, dstbufs, rowbufs, agg, se, sg, ss):
    c = lax.axis_index("c")
    s = lax.axis_index("s")

    @pl.when(s < WB_TILES)
    def _():
        pltpu.sync_copy(zeros_hbm, agg.at[pl.ds(s * WB_ROWS, WB_ROWS)])

    ebase = pl.multiple_of(s * E_PER, 8)

    def eload(j, q):
        pltpu.async_copy(packed_hbm.at[pl.ds(ebase + j * CH, CH)],
                         ebufs[q], se[q])

    def wait_eload(j, q):
        pltpu.make_async_copy(packed_hbm.at[pl.ds(ebase + j * CH, CH)],
                              ebufs[q], se[q]).wait()

    def unpack(q):
        for k in range(CH // 16):
            sl = pl.ds(k * 16, 16)
            v = ebufs[q][sl]
            srcv = v & 0x3FFF
            dstv = lax.shift_right_logical(v, 14) & 0x3FFF
            etv = lax.shift_right_logical(v, 28)
            idxbufs[q][sl] = (etv * NC + c) * N + srcv
            dstbufs[q][sl] = dstv

    def gather(j, q):
        pltpu.async_copy(table_hbm.at[idxbufs[q]], rowbufs[q], sg[q])

    def wait_gather(j, q):
        pltpu.make_async_copy(table_hbm.at[idxbufs[q]], rowbufs[q],
                              sg[q]).wait()

    def scatter(j, q):
        pltpu.async_copy(rowbufs[q], agg.at[dstbufs[q]], ss[q], add=True)

    def wait_scatter(j, q):
        pltpu.make_async_copy(rowbufs[q], agg.at[dstbufs[q]], ss[q]).wait()

    for j in range(4):
        eload(j, j)
    plsc.subcore_barrier()

    for j in range(2):
        wait_eload(j, j)
        unpack(j)
        gather(j, j)

    def quad(t, _):
        for q in range(4):
            j = t * 4 + q
            wait_gather(j, q)
            scatter(j, q)
            p2 = (q + 2) % 4

            @pl.when(j >= 2)
            def _():
                wait_scatter(j - 2, p2)

            @pl.when(j + 2 < NCHUNK)
            def _():
                wait_eload(j + 2, p2)
                unpack(p2)
                gather(j + 2, p2)

            @pl.when(j + 4 < NCHUNK)
            def _():
                eload(j + 4, q)
        return ()

    lax.fori_loop(0, NCHUNK // 4, quad, (), unroll=False)
    j = NCHUNK - 1
    wait_gather(j, 0)
    scatter(j, 0)
    wait_scatter(j - 2, 2)
    wait_scatter(j - 1, 3)
    wait_scatter(j, 0)
    plsc.subcore_barrier()

    @pl.when(s < WB_TILES)
    def _():
        sl_out = pl.ds(s * WB_ROWS, WB_ROWS)
        pltpu.sync_copy(agg.at[sl_out], out_hbm.at[c, sl_out])


def _sc_aggregate(table, packed, zeros):
    mesh = plsc.VectorSubcoreMesh(core_axis_name="c", subcore_axis_name="s")
    kern = pl.kernel(
        _sc_body,
        out_type=jax.ShapeDtypeStruct((NC, N, HALF), jnp.float32),
        mesh=mesh,
        scratch_types=[
            [pltpu.VMEM((CH,), jnp.int32) for _ in range(4)],
            [pltpu.VMEM((CH,), jnp.int32) for _ in range(4)],
            [pltpu.VMEM((CH,), jnp.int32) for _ in range(4)],
            [pltpu.VMEM((CH, HALF), jnp.float32) for _ in range(4)],
            pltpu.VMEM_SHARED((N, HALF), jnp.float32),
            [pltpu.SemaphoreType.DMA for _ in range(4)],
            [pltpu.SemaphoreType.DMA for _ in range(4)],
            [pltpu.SemaphoreType.DMA for _ in range(4)],
        ],
    )
    return kern(table, packed, zeros)


def _final_kernel(curr_ref, agg_ref, out_ref):
    out_ref[...] = curr_ref[...] + jnp.concatenate(
        [agg_ref[0], agg_ref[1]], axis=1)


def _finalize(curr, agg):
    bn = 2000
    return pl.pallas_call(
        _final_kernel,
        grid=(N // bn,),
        in_specs=[
            pl.BlockSpec((bn, OUT_DIM), lambda i: (i, 0)),
            pl.BlockSpec((NC, bn, HALF), lambda i: (0, i, 0)),
        ],
        out_specs=pl.BlockSpec((bn, OUT_DIM), lambda i: (i, 0)),
        out_shape=jax.ShapeDtypeStruct((N, OUT_DIM), jnp.float32),
    )(curr, agg)


def kernel(x, edge_index, edge_type, weight, w_comp, self_loop_weight):
    src = edge_index[0]
    dst = edge_index[1]
    packed = src | (dst << 14) | (edge_type << 28)
    zeros = jnp.zeros((WB_ROWS, HALF), jnp.float32)

    table, curr = _build_table(x, weight, w_comp, self_loop_weight)
    agg = _sc_aggregate(table.reshape(NUM_RELS * NC * N, HALF),
                        packed, zeros)
    return _finalize(curr, agg)


# speedup vs baseline: 62.0775x; 1.0551x over previous
import jax
import jax.numpy as jnp
from jax import lax
from jax.experimental import pallas as pl
from jax.experimental.pallas import tpu as pltpu
from jax.experimental.pallas import tpu_sc as plsc

N = 10000
E = 160000
IN_DIM = 256
OUT_DIM = 256
NUM_RELS = 16
NUM_BASES = 4

NC = 2
NS = 16
HALF = OUT_DIM // 2

BN = 2000
NB = N // BN

E_PER = E // NS
CH = 80
NCHUNK = E_PER // CH

WB_TILES = 10
WB_ROWS = N // WB_TILES


def _table_kernel(wc_ref, x_ref, w_ref, slw_ref, out_ref, curr_ref, wall_ref):
    nb = pl.program_id(0)
    r = pl.program_id(1)

    @pl.when(nb == 0)
    def _():
        w = (wc_ref[r, 0] * w_ref[0] + wc_ref[r, 1] * w_ref[1]
             + wc_ref[r, 2] * w_ref[2] + wc_ref[r, 3] * w_ref[3])
        wall_ref[r] = w.astype(jnp.bfloat16)

    xb = x_ref[...].astype(jnp.bfloat16)
    zz = jnp.dot(xb, wall_ref[r], preferred_element_type=jnp.float32)
    out_ref[0] = zz[:, :HALF]
    out_ref[1] = zz[:, HALF:]

    @pl.when(r == 0)
    def _():
        cur = jnp.dot(xb, slw_ref[...].astype(jnp.bfloat16),
                      preferred_element_type=jnp.float32)
        curr_ref[0] = cur[:, :HALF]
        curr_ref[1] = cur[:, HALF:]


def _build_table(x, weight, w_comp, self_loop_weight):
    return pl.pallas_call(
        _table_kernel,
        grid=(NB, NUM_RELS),
        in_specs=[
            pl.BlockSpec(memory_space=pltpu.SMEM),
            pl.BlockSpec((BN, IN_DIM), lambda nb, r: (nb, 0)),
            pl.BlockSpec((NUM_BASES, IN_DIM, OUT_DIM), lambda nb, r: (0, 0, 0)),
            pl.BlockSpec((IN_DIM, OUT_DIM), lambda nb, r: (0, 0)),
        ],
        out_specs=[
            pl.BlockSpec((NC, BN, HALF), lambda nb, r: (r, nb, 0)),
            pl.BlockSpec((NC, BN, HALF), lambda nb, r: (0, nb, 0)),
        ],
        out_shape=[
            jax.ShapeDtypeStruct((NUM_RELS * NC, N, HALF), jnp.float32),
            jax.ShapeDtypeStruct((NC, N, HALF), jnp.float32),
        ],
        scratch_shapes=[
            pltpu.VMEM((NUM_RELS, IN_DIM, OUT_DIM), jnp.bfloat16),
        ],
    )(w_comp, x, weight, self_loop_weight)


def _sc_body(table_hbm, packed_hbm, curr_hbm, out_hbm,
             ebufs, idxbufs, dstbufs, rowbufs, agg, se, sg, ss):
    c = lax.axis_index("c")
    s = lax.axis_index("s")

    @pl.when(s < WB_TILES)
    def _():
        sl_init = pl.ds(s * WB_ROWS, WB_ROWS)
        pltpu.sync_copy(curr_hbm.at[c, sl_init], agg.at[sl_init])

    ebase = pl.multiple_of(s * E_PER, 8)

    def eload(j, q):
        pltpu.async_copy(packed_hbm.at[pl.ds(ebase + j * CH, CH)],
                         ebufs[q], se[q])

    def wait_eload(j, q):
        pltpu.make_async_copy(packed_hbm.at[pl.ds(ebase + j * CH, CH)],
                              ebufs[q], se[q]).wait()

    def unpack(q):
        for k in range(CH // 16):
            sl = pl.ds(k * 16, 16)
            v = ebufs[q][sl]
            srcv = v & 0x3FFF
            dstv = lax.shift_right_logical(v, 14) & 0x3FFF
            etv = lax.shift_right_logical(v, 28)
            idxbufs[q][sl] = (etv * NC + c) * N + srcv
            dstbufs[q][sl] = dstv

    def gather(j, q):
        pltpu.async_copy(table_hbm.at[idxbufs[q]], rowbufs[q], sg[q])

    def wait_gather(j, q):
        pltpu.make_async_copy(table_hbm.at[idxbufs[q]], rowbufs[q],
                              sg[q]).wait()

    def scatter(j, q):
        pltpu.async_copy(rowbufs[q], agg.at[dstbufs[q]], ss[q], add=True)

    def wait_scatter(j, q):
        pltpu.make_async_copy(rowbufs[q], agg.at[dstbufs[q]], ss[q]).wait()

    for j in range(4):
        eload(j, j)
    plsc.subcore_barrier()

    for j in range(2):
        wait_eload(j, j)
        unpack(j)
        gather(j, j)

    def quad(t, _):
        for q in range(4):
            j = t * 4 + q
            wait_gather(j, q)
            scatter(j, q)
            p2 = (q + 2) % 4

            @pl.when(j >= 2)
            def _():
                wait_scatter(j - 2, p2)

            @pl.when(j + 2 < NCHUNK)
            def _():
                wait_eload(j + 2, p2)
                unpack(p2)
                gather(j + 2, p2)

            @pl.when(j + 4 < NCHUNK)
            def _():
                eload(j + 4, q)
        return ()

    lax.fori_loop(0, NCHUNK // 4, quad, (), unroll=False)
    j = NCHUNK - 1
    wait_gather(j, 0)
    scatter(j, 0)
    wait_scatter(j - 2, 2)
    wait_scatter(j - 1, 3)
    wait_scatter(j, 0)
    plsc.subcore_barrier()

    @pl.when(s < WB_TILES)
    def _():
        sl_out = pl.ds(s * WB_ROWS, WB_ROWS)
        pltpu.sync_copy(agg.at[sl_out],
                        out_hbm.at[sl_out, pl.ds(c * HALF, HALF)])


def _sc_aggregate(table, packed, curr):
    mesh = plsc.VectorSubcoreMesh(core_axis_name="c", subcore_axis_name="s")
    kern = pl.kernel(
        _sc_body,
        out_type=jax.ShapeDtypeStruct((N, OUT_DIM), jnp.float32),
        mesh=mesh,
        scratch_types=[
            [pltpu.VMEM((CH,), jnp.int32) for _ in range(4)],
            [pltpu.VMEM((CH,), jnp.int32) for _ in range(4)],
            [pltpu.VMEM((CH,), jnp.int32) for _ in range(4)],
            [pltpu.VMEM((CH, HALF), jnp.float32) for _ in range(4)],
            pltpu.VMEM_SHARED((N, HALF), jnp.float32),
            [pltpu.SemaphoreType.DMA for _ in range(4)],
            [pltpu.SemaphoreType.DMA for _ in range(4)],
            [pltpu.SemaphoreType.DMA for _ in range(4)],
        ],
    )
    return kern(table, packed, curr)


def kernel(x, edge_index, edge_type, weight, w_comp, self_loop_weight):
    src = edge_index[0]
    dst = edge_index[1]
    packed = src | (dst << 14) | (edge_type << 28)

    table, curr = _build_table(x, weight, w_comp, self_loop_weight)
    return _sc_aggregate(table.reshape(NUM_RELS * NC * N, HALF),
                         packed, curr)


# speedup vs baseline: 63.6403x; 1.0252x over previous
import jax
import jax.numpy as jnp
from jax import lax
from jax.experimental import pallas as pl
from jax.experimental.pallas import tpu as pltpu
from jax.experimental.pallas import tpu_sc as plsc

N = 10000
E = 160000
IN_DIM = 256
OUT_DIM = 256
NUM_RELS = 16
NUM_BASES = 4

NC = 2
NS = 16
HALF = OUT_DIM // 2

BN = 2000
NB = N // BN

E_PER = E // NS
CH = 80
NCHUNK = E_PER // CH

WB_TILES = 10
WB_ROWS = N // WB_TILES

EROWS = E // 128


def _table_kernel(wc_ref, x_ref, w_ref, slw_ref, ei_ref, et_ref,
                  out_ref, curr_ref, packed_ref, wall_ref, xb_ref):
    nb = pl.program_id(0)
    r = pl.program_id(1)

    @pl.when(nb == 0)
    def _():
        w = (wc_ref[r, 0] * w_ref[0] + wc_ref[r, 1] * w_ref[1]
             + wc_ref[r, 2] * w_ref[2] + wc_ref[r, 3] * w_ref[3])
        wall_ref[r] = w.astype(jnp.bfloat16)

    @pl.when(r == 0)
    def _():
        xb_ref[...] = x_ref[...].astype(jnp.bfloat16)

    @pl.when((nb == 0) & (r == 1))
    def _():
        packed_ref[...] = (ei_ref[0] | (ei_ref[1] << 14)
                           | (et_ref[...] << 28))

    zz = jnp.dot(xb_ref[...], wall_ref[r], preferred_element_type=jnp.float32)
    out_ref[0] = zz[:, :HALF]
    out_ref[1] = zz[:, HALF:]

    @pl.when(r == 0)
    def _():
        cur = jnp.dot(xb_ref[...], slw_ref[...].astype(jnp.bfloat16),
                      preferred_element_type=jnp.float32)
        curr_ref[0] = cur[:, :HALF]
        curr_ref[1] = cur[:, HALF:]


def _build_table(x, weight, w_comp, self_loop_weight, edge_index, edge_type):
    return pl.pallas_call(
        _table_kernel,
        grid=(NB, NUM_RELS),
        in_specs=[
            pl.BlockSpec(memory_space=pltpu.SMEM),
            pl.BlockSpec((BN, IN_DIM), lambda nb, r: (nb, 0)),
            pl.BlockSpec((NUM_BASES, IN_DIM, OUT_DIM), lambda nb, r: (0, 0, 0)),
            pl.BlockSpec((IN_DIM, OUT_DIM), lambda nb, r: (0, 0)),
            pl.BlockSpec((2, EROWS, 128), lambda nb, r: (0, 0, 0)),
            pl.BlockSpec((EROWS, 128), lambda nb, r: (0, 0)),
        ],
        out_specs=[
            pl.BlockSpec((NC, BN, HALF), lambda nb, r: (r, nb, 0)),
            pl.BlockSpec((NC, BN, HALF), lambda nb, r: (0, nb, 0)),
            pl.BlockSpec((EROWS, 128), lambda nb, r: (0, 0)),
        ],
        out_shape=[
            jax.ShapeDtypeStruct((NUM_RELS * NC, N, HALF), jnp.float32),
            jax.ShapeDtypeStruct((NC, N, HALF), jnp.float32),
            jax.ShapeDtypeStruct((EROWS, 128), jnp.int32),
        ],
        scratch_shapes=[
            pltpu.VMEM((NUM_RELS, IN_DIM, OUT_DIM), jnp.bfloat16),
            pltpu.VMEM((BN, IN_DIM), jnp.bfloat16),
        ],
    )(w_comp, x, weight, self_loop_weight,
      edge_index.reshape(2, EROWS, 128), edge_type.reshape(EROWS, 128))


def _sc_body(table_hbm, packed_hbm, curr_hbm, out_hbm,
             ebufs, idxbufs, dstbufs, rowbufs, agg, se, sg, ss):
    c = lax.axis_index("c")
    s = lax.axis_index("s")

    @pl.when(s < WB_TILES)
    def _():
        sl_init = pl.ds(s * WB_ROWS, WB_ROWS)
        pltpu.sync_copy(curr_hbm.at[c, sl_init], agg.at[sl_init])

    ebase = pl.multiple_of(s * E_PER, 8)

    def eload(j, q):
        pltpu.async_copy(packed_hbm.at[pl.ds(ebase + j * CH, CH)],
                         ebufs[q], se[q])

    def wait_eload(j, q):
        pltpu.make_async_copy(packed_hbm.at[pl.ds(ebase + j * CH, CH)],
                              ebufs[q], se[q]).wait()

    def unpack(q):
        for k in range(CH // 16):
            sl = pl.ds(k * 16, 16)
            v = ebufs[q][sl]
            srcv = v & 0x3FFF
            dstv = lax.shift_right_logical(v, 14) & 0x3FFF
            etv = lax.shift_right_logical(v, 28)
            idxbufs[q][sl] = (etv * NC + c) * N + srcv
            dstbufs[q][sl] = dstv

    def gather(j, q):
        pltpu.async_copy(table_hbm.at[idxbufs[q]], rowbufs[q], sg[q])

    def wait_gather(j, q):
        pltpu.make_async_copy(table_hbm.at[idxbufs[q]], rowbufs[q],
                              sg[q]).wait()

    def scatter(j, q):
        pltpu.async_copy(rowbufs[q], agg.at[dstbufs[q]], ss[q], add=True)

    def wait_scatter(j, q):
        pltpu.make_async_copy(rowbufs[q], agg.at[dstbufs[q]], ss[q]).wait()

    for j in range(4):
        eload(j, j)
    plsc.subcore_barrier()

    for j in range(2):
        wait_eload(j, j)
        unpack(j)
        gather(j, j)

    def quad(t, _):
        for q in range(4):
            j = t * 4 + q
            wait_gather(j, q)
            scatter(j, q)
            p2 = (q + 2) % 4

            @pl.when(j >= 2)
            def _():
                wait_scatter(j - 2, p2)

            @pl.when(j + 2 < NCHUNK)
            def _():
                wait_eload(j + 2, p2)
                unpack(p2)
                gather(j + 2, p2)

            @pl.when(j + 4 < NCHUNK)
            def _():
                eload(j + 4, q)
        return ()

    lax.fori_loop(0, NCHUNK // 4, quad, (), unroll=False)
    j = NCHUNK - 1
    wait_gather(j, 0)
    scatter(j, 0)
    wait_scatter(j - 2, 2)
    wait_scatter(j - 1, 3)
    wait_scatter(j, 0)
    plsc.subcore_barrier()

    @pl.when(s < WB_TILES)
    def _():
        sl_out = pl.ds(s * WB_ROWS, WB_ROWS)
        pltpu.sync_copy(agg.at[sl_out],
                        out_hbm.at[sl_out, pl.ds(c * HALF, HALF)])


def _sc_aggregate(table, packed, curr):
    mesh = plsc.VectorSubcoreMesh(core_axis_name="c", subcore_axis_name="s")
    kern = pl.kernel(
        _sc_body,
        out_type=jax.ShapeDtypeStruct((N, OUT_DIM), jnp.float32),
        mesh=mesh,
        scratch_types=[
            [pltpu.VMEM((CH,), jnp.int32) for _ in range(4)],
            [pltpu.VMEM((CH,), jnp.int32) for _ in range(4)],
            [pltpu.VMEM((CH,), jnp.int32) for _ in range(4)],
            [pltpu.VMEM((CH, HALF), jnp.float32) for _ in range(4)],
            pltpu.VMEM_SHARED((N, HALF), jnp.float32),
            [pltpu.SemaphoreType.DMA for _ in range(4)],
            [pltpu.SemaphoreType.DMA for _ in range(4)],
            [pltpu.SemaphoreType.DMA for _ in range(4)],
        ],
    )
    return kern(table, packed, curr)


def kernel(x, edge_index, edge_type, weight, w_comp, self_loop_weight):
    table, curr, packed = _build_table(x, weight, w_comp, self_loop_weight,
                                       edge_index, edge_type)
    return _sc_aggregate(table.reshape(NUM_RELS * NC * N, HALF),
                         packed.reshape(E), curr)
